# Initial kernel scaffold; baseline (speedup 1.0000x reference)
import jax
import jax.numpy as jnp
from jax.experimental import pallas as pl


def kernel(hs_0, hs_1, hs_2, hs_3, hs_4, hs_5, hs_6, hs_7, hs_8, hs_9, hs_10, hs_11, hs_12, layer_weights):
    raise NotImplementedError("write your pallas kernel here")


# speedup vs baseline: 7.6963x; 7.6963x over previous
import jax
import jax.numpy as jnp
from jax.experimental import pallas as pl
from jax.experimental.pallas import tpu as pltpu

_LAYER_START = 4
_ROW_TILE = 256


def _wlp_kernel(w_ref, *refs):
    layer_refs = refs[:-1]
    out_ref = refs[-1]
    n = len(layer_refs)
    s = w_ref[0]
    for i in range(1, n):
        s = s + w_ref[i]
    inv = 1.0 / s
    acc = (w_ref[0] * inv) * layer_refs[0][...].astype(jnp.float32)
    for i in range(1, n):
        acc = acc + (w_ref[i] * inv) * layer_refs[i][...].astype(jnp.float32)
    out_ref[...] = acc


def kernel(hs_0, hs_1, hs_2, hs_3, hs_4, hs_5, hs_6, hs_7, hs_8, hs_9,
           hs_10, hs_11, hs_12, layer_weights):
    layers = [hs_0, hs_1, hs_2, hs_3, hs_4, hs_5, hs_6, hs_7, hs_8, hs_9,
              hs_10, hs_11, hs_12][_LAYER_START:]
    B, S, H = layers[0].shape
    R = B * S
    flat = [x.reshape(R, H) for x in layers]

    tm = _ROW_TILE
    while R % tm != 0:
        tm //= 2
    num_tiles = R // tm

    out2d = pl.pallas_call(
        _wlp_kernel,
        out_shape=jax.ShapeDtypeStruct((R, H), jnp.float32),
        grid_spec=pltpu.PrefetchScalarGridSpec(
            num_scalar_prefetch=0,
            grid=(num_tiles,),
            in_specs=[pl.BlockSpec(memory_space=pltpu.MemorySpace.SMEM)]
            + [pl.BlockSpec((tm, H), lambda rt: (rt, 0)) for _ in flat],
            out_specs=pl.BlockSpec((tm, H), lambda rt: (rt, 0)),
        ),
        compiler_params=pltpu.CompilerParams(
            dimension_semantics=("parallel",)),
    )(layer_weights.astype(jnp.float32), *flat)

    return out2d.reshape(B, S, H)
